# baseline (speedup 1.0000x reference)
import jax
import jax.numpy as jnp
from jax import lax
from jax.experimental import pallas as pl
from jax.experimental.pallas import tpu as pltpu

C = 256
T = 10000
THRESH = 0.6


def _fused(d_ref, wg_ref, bg_ref, wc_ref, bc_ref, out_ref):
    d = d_ref[...]
    inv_t = jnp.float32(1.0 / T)

    d2 = d * d
    s1 = jnp.sum(d, axis=1, keepdims=True)
    s2 = jnp.sum(d2, axis=1, keepdims=True)
    s3 = jnp.sum(d2 * d, axis=1, keepdims=True)
    s4 = jnp.sum(d2 * d2, axis=1, keepdims=True)

    g = lax.dot_general(d, d, (((1,), (1,)), ((), ())),
                        preferred_element_type=jnp.float32)

    mu = s1 * inv_t
    mu_r = jnp.transpose(mu)

    normsq = jnp.maximum(s2 - (jnp.float32(T) * mu) * mu, 0.0)
    inv_norm = lax.rsqrt(jnp.maximum(normsq, jnp.float32(1e-12)))
    corr = (g - (jnp.float32(T) * mu) * mu_r) * inv_norm * jnp.transpose(inv_norm)
    corr = jnp.clip(corr, -1.0, 1.0)

    row = lax.broadcasted_iota(jnp.int32, (C, C), 0)
    col = lax.broadcasted_iota(jnp.int32, (C, C), 1)
    on_diag = row == col
    ac = jnp.abs(corr)
    mask = (ac >= jnp.float32(THRESH)) & (~on_diag)
    w = jnp.clip(ac, 1e-6, 0.99)
    a = jnp.where(mask, w, 0.0) + jnp.where(on_diag, 1.0, 0.0)

    m2 = s2 * inv_t - mu * mu
    m3 = s3 * inv_t - 3.0 * mu * (s2 * inv_t) + 2.0 * mu * mu * mu
    m4 = (s4 * inv_t - 4.0 * mu * (s3 * inv_t)
          + 6.0 * (mu * mu) * (s2 * inv_t) - 3.0 * (mu * mu) * (mu * mu))
    m2s = jnp.maximum(m2, jnp.float32(1e-12))
    inv_m2s = lax.rsqrt(m2s)
    skew = m3 * inv_m2s * inv_m2s * inv_m2s
    kurt = m4 * (inv_m2s * inv_m2s) * (inv_m2s * inv_m2s) - 3.0
    x = jnp.concatenate([mu, m2, skew, kurt], axis=1)

    agg = jnp.dot(a, x, preferred_element_type=jnp.float32)
    h = jnp.dot(agg, wg_ref[...], preferred_element_type=jnp.float32)
    h = jnp.maximum(h + bg_ref[...], 0.0)
    ge = jnp.sum(h, axis=0, keepdims=True)
    logits = jnp.dot(ge, wc_ref[...], preferred_element_type=jnp.float32)
    out_ref[...] = logits + bc_ref[...]


def kernel(data, W_gfc, b_gfc, W_cls, b_cls):
    out = pl.pallas_call(
        _fused,
        out_shape=jax.ShapeDtypeStruct((1, 2), jnp.float32),
    )(data, W_gfc, b_gfc.reshape(1, -1), W_cls, b_cls.reshape(1, -1))
    return out
